# baseline (speedup 1.0000x reference)
import functools

import jax
import jax.numpy as jnp
from jax import lax
from jax.experimental import pallas as pl
from jax.experimental.pallas import tpu as pltpu
from jax.experimental.pallas import tpu_sc as plsc

NUM_CLASS = 100000
CTX_DIM = 512
N_CLS_CTX = 4
B = 4096
N_TOK = 77

NUM_CORES = 2
NUM_SUBCORES = 16
NUM_WORKERS = NUM_CORES * NUM_SUBCORES
PER_W = B // NUM_WORKERS
CHUNK = 16
N_CHUNKS = PER_W // CHUNK


@functools.partial(
    pl.kernel,
    out_type=[jax.ShapeDtypeStruct((B, N_CLS_CTX, CTX_DIM), jnp.float32)] * 3,
    mesh=plsc.VectorSubcoreMesh(core_axis_name="c", subcore_axis_name="s"),
    scratch_types=[
        pltpu.VMEM((PER_W,), jnp.int32),
        pltpu.VMEM((CHUNK, N_CLS_CTX, CTX_DIM), jnp.float32),
        pltpu.VMEM((CHUNK, N_CLS_CTX, CTX_DIM), jnp.float32),
        pltpu.VMEM((CHUNK, N_CLS_CTX, CTX_DIM), jnp.float32),
        pltpu.SemaphoreType.DMA,
        pltpu.SemaphoreType.DMA,
    ],
)
def _gather_sc(label_hbm, t1_hbm, t2_hbm, t3_hbm, c1_out, c2_out, c3_out,
               idx_v, r1, r2, r3, gsem, ssem):
    wid = lax.axis_index("s") * NUM_CORES + lax.axis_index("c")
    base = wid * PER_W
    pltpu.sync_copy(label_hbm.at[pl.ds(base, PER_W)], idx_v)

    def chunk_body(c, carry):
        off = c * CHUNK
        idx_c = idx_v.at[pl.ds(off, CHUNK)]
        g1 = pltpu.async_copy(t1_hbm.at[idx_c], r1, gsem)
        g2 = pltpu.async_copy(t2_hbm.at[idx_c], r2, gsem)
        g3 = pltpu.async_copy(t3_hbm.at[idx_c], r3, gsem)
        g1.wait()
        g2.wait()
        g3.wait()
        s1 = pltpu.async_copy(r1, c1_out.at[pl.ds(base + off, CHUNK)], ssem)
        s2 = pltpu.async_copy(r2, c2_out.at[pl.ds(base + off, CHUNK)], ssem)
        s3 = pltpu.async_copy(r3, c3_out.at[pl.ds(base + off, CHUNK)], ssem)
        s1.wait()
        s2.wait()
        s3.wait()
        return carry

    lax.fori_loop(0, N_CHUNKS, chunk_body, 0)


BLK = 16


def _assemble(c1, c2, c3, pre, s1, s2, suf, out):
    out[:, 0:5] = jnp.broadcast_to(pre[...], (BLK, 5, CTX_DIM))
    out[:, 5:9] = c1[...]
    out[:, 9:11] = jnp.broadcast_to(s1[...], (BLK, 2, CTX_DIM))
    out[:, 11:15] = c2[...]
    out[:, 15:18] = jnp.broadcast_to(s2[...], (BLK, 3, CTX_DIM))
    out[:, 18:22] = c3[...]
    out[:, 22:77] = jnp.broadcast_to(suf[...], (BLK, 55, CTX_DIM))


@jax.jit
def _prompt_concat(label, cls_ctx, cls_ctx2, cls_ctx3, token_prefix,
                   token_suffix_1, token_suffix_2, token_suffix):
    c1, c2, c3 = _gather_sc(label.astype(jnp.int32), cls_ctx, cls_ctx2,
                            cls_ctx3)
    cblk = pl.BlockSpec((BLK, N_CLS_CTX, CTX_DIM), lambda i: (i, 0, 0))
    return pl.pallas_call(
        _assemble,
        grid=(B // BLK,),
        in_specs=[
            cblk,
            cblk,
            cblk,
            pl.BlockSpec((1, 5, CTX_DIM), lambda i: (0, 0, 0)),
            pl.BlockSpec((1, 2, CTX_DIM), lambda i: (0, 0, 0)),
            pl.BlockSpec((1, 3, CTX_DIM), lambda i: (0, 0, 0)),
            pl.BlockSpec((1, 55, CTX_DIM), lambda i: (0, 0, 0)),
        ],
        out_specs=pl.BlockSpec((BLK, N_TOK, CTX_DIM), lambda i: (i, 0, 0)),
        out_shape=jax.ShapeDtypeStruct((B, N_TOK, CTX_DIM), jnp.float32),
        compiler_params=pltpu.CompilerParams(
            dimension_semantics=("arbitrary",)),
    )(c1, c2, c3, token_prefix, token_suffix_1, token_suffix_2, token_suffix)


def kernel(label, cls_ctx, cls_ctx2, cls_ctx3, token_prefix, token_suffix_1,
           token_suffix_2, token_suffix):
    return _prompt_concat(label, cls_ctx, cls_ctx2, cls_ctx3, token_prefix,
                          token_suffix_1, token_suffix_2, token_suffix)


# speedup vs baseline: 1.0018x; 1.0018x over previous
import functools

import jax
import jax.numpy as jnp
from jax import lax
from jax.experimental import pallas as pl
from jax.experimental.pallas import tpu as pltpu
from jax.experimental.pallas import tpu_sc as plsc

NUM_CLASS = 100000
CTX_DIM = 512
N_CLS_CTX = 4
B = 4096
N_TOK = 77

NUM_CORES = 2
NUM_SUBCORES = 16
NUM_WORKERS = NUM_CORES * NUM_SUBCORES
PER_W = B // NUM_WORKERS
CHUNK = 16
N_CHUNKS = PER_W // CHUNK


@functools.partial(
    pl.kernel,
    out_type=[jax.ShapeDtypeStruct((B, N_CLS_CTX, CTX_DIM), jnp.float32)] * 3,
    mesh=plsc.VectorSubcoreMesh(core_axis_name="c", subcore_axis_name="s"),
    scratch_types=[
        pltpu.VMEM((PER_W,), jnp.int32),
        pltpu.VMEM((CHUNK, N_CLS_CTX, CTX_DIM), jnp.float32),
        pltpu.VMEM((CHUNK, N_CLS_CTX, CTX_DIM), jnp.float32),
        pltpu.VMEM((CHUNK, N_CLS_CTX, CTX_DIM), jnp.float32),
        pltpu.SemaphoreType.DMA,
        pltpu.SemaphoreType.DMA,
    ],
)
def _gather_sc(label_hbm, t1_hbm, t2_hbm, t3_hbm, c1_out, c2_out, c3_out,
               idx_v, r1, r2, r3, gsem, ssem):
    wid = lax.axis_index("s") * NUM_CORES + lax.axis_index("c")
    base = wid * PER_W
    pltpu.sync_copy(label_hbm.at[pl.ds(base, PER_W)], idx_v)

    def chunk_body(c, carry):
        off = c * CHUNK
        idx_c = idx_v.at[pl.ds(off, CHUNK)]
        g1 = pltpu.async_copy(t1_hbm.at[idx_c], r1, gsem)
        g2 = pltpu.async_copy(t2_hbm.at[idx_c], r2, gsem)
        g3 = pltpu.async_copy(t3_hbm.at[idx_c], r3, gsem)
        g1.wait()
        g2.wait()
        g3.wait()
        s1 = pltpu.async_copy(r1, c1_out.at[pl.ds(base + off, CHUNK)], ssem)
        s2 = pltpu.async_copy(r2, c2_out.at[pl.ds(base + off, CHUNK)], ssem)
        s3 = pltpu.async_copy(r3, c3_out.at[pl.ds(base + off, CHUNK)], ssem)
        s1.wait()
        s2.wait()
        s3.wait()
        return carry

    lax.fori_loop(0, N_CHUNKS, chunk_body, 0)


BLK = 16


def _assemble(c1, c2, c3, pre, s1, s2, suf, out):
    out[:, 0:5] = jnp.broadcast_to(pre[...], (BLK, 5, CTX_DIM))
    out[:, 5:9] = c1[...]
    out[:, 9:11] = jnp.broadcast_to(s1[...], (BLK, 2, CTX_DIM))
    out[:, 11:15] = c2[...]
    out[:, 15:18] = jnp.broadcast_to(s2[...], (BLK, 3, CTX_DIM))
    out[:, 18:22] = c3[...]
    out[:, 22:77] = jnp.broadcast_to(suf[...], (BLK, 55, CTX_DIM))


def _prompt_concat(label, cls_ctx, cls_ctx2, cls_ctx3, token_prefix,
                   token_suffix_1, token_suffix_2, token_suffix):
    c1, c2, c3 = _gather_sc(label.astype(jnp.int32), cls_ctx, cls_ctx2,
                            cls_ctx3)
    cblk = pl.BlockSpec((BLK, N_CLS_CTX, CTX_DIM), lambda i: (i, 0, 0))
    return pl.pallas_call(
        _assemble,
        grid=(B // BLK,),
        in_specs=[
            cblk,
            cblk,
            cblk,
            pl.BlockSpec((1, 5, CTX_DIM), lambda i: (0, 0, 0)),
            pl.BlockSpec((1, 2, CTX_DIM), lambda i: (0, 0, 0)),
            pl.BlockSpec((1, 3, CTX_DIM), lambda i: (0, 0, 0)),
            pl.BlockSpec((1, 55, CTX_DIM), lambda i: (0, 0, 0)),
        ],
        out_specs=pl.BlockSpec((BLK, N_TOK, CTX_DIM), lambda i: (i, 0, 0)),
        out_shape=jax.ShapeDtypeStruct((B, N_TOK, CTX_DIM), jnp.float32),
        compiler_params=pltpu.CompilerParams(
            dimension_semantics=("arbitrary",)),
    )(c1, c2, c3, token_prefix, token_suffix_1, token_suffix_2, token_suffix)


def kernel(label, cls_ctx, cls_ctx2, cls_ctx3, token_prefix, token_suffix_1,
           token_suffix_2, token_suffix):
    return _prompt_concat(label, cls_ctx, cls_ctx2, cls_ctx3, token_prefix,
                          token_suffix_1, token_suffix_2, token_suffix)


# speedup vs baseline: 2.0847x; 2.0810x over previous
import functools

import jax
import jax.numpy as jnp
from jax import lax
from jax.experimental import pallas as pl
from jax.experimental.pallas import tpu as pltpu
from jax.experimental.pallas import tpu_sc as plsc

NUM_CLASS = 100000
CTX_DIM = 512
N_CLS_CTX = 4
B = 4096
N_TOK = 77

NUM_CORES = 2
NUM_SUBCORES = 16
NUM_WORKERS = NUM_CORES * NUM_SUBCORES
PER_W = B // NUM_WORKERS
CHUNK = 16
N_CHUNKS = PER_W // CHUNK


@functools.partial(
    pl.kernel,
    out_type=[jax.ShapeDtypeStruct((B, N_CLS_CTX, CTX_DIM), jnp.float32)] * 3,
    mesh=plsc.VectorSubcoreMesh(core_axis_name="c", subcore_axis_name="s"),
    scratch_types=[
        pltpu.VMEM((PER_W,), jnp.int32),
        pltpu.VMEM((CHUNK, N_CLS_CTX, CTX_DIM), jnp.float32),
        pltpu.VMEM((CHUNK, N_CLS_CTX, CTX_DIM), jnp.float32),
        pltpu.VMEM((CHUNK, N_CLS_CTX, CTX_DIM), jnp.float32),
        pltpu.SemaphoreType.DMA,
        pltpu.SemaphoreType.DMA,
    ],
)
def _gather_sc(label_hbm, t1_hbm, t2_hbm, t3_hbm, c1_out, c2_out, c3_out,
               idx_v, r1, r2, r3, gsem, ssem):
    wid = lax.axis_index("s") * NUM_CORES + lax.axis_index("c")
    base = wid * PER_W
    pltpu.sync_copy(label_hbm.at[pl.ds(base, PER_W)], idx_v)

    def chunk_body(c, carry):
        off = c * CHUNK
        idx_c = idx_v.at[pl.ds(off, CHUNK)]
        g1 = pltpu.async_copy(t1_hbm.at[idx_c], r1, gsem)
        g2 = pltpu.async_copy(t2_hbm.at[idx_c], r2, gsem)
        g3 = pltpu.async_copy(t3_hbm.at[idx_c], r3, gsem)
        g1.wait()
        g2.wait()
        g3.wait()
        s1 = pltpu.async_copy(r1, c1_out.at[pl.ds(base + off, CHUNK)], ssem)
        s2 = pltpu.async_copy(r2, c2_out.at[pl.ds(base + off, CHUNK)], ssem)
        s3 = pltpu.async_copy(r3, c3_out.at[pl.ds(base + off, CHUNK)], ssem)
        s1.wait()
        s2.wait()
        s3.wait()
        return carry

    lax.fori_loop(0, N_CHUNKS, chunk_body, 0)


BLK = 16


def _assemble(c1, c2, c3, pre, s1, s2, suf, out):
    for t in range(5):
        out[t] = jnp.broadcast_to(pre[0, t], (BLK, CTX_DIM))
    for j in range(N_CLS_CTX):
        out[5 + j] = c1[:, j, :]
    for t in range(2):
        out[9 + t] = jnp.broadcast_to(s1[0, t], (BLK, CTX_DIM))
    for j in range(N_CLS_CTX):
        out[11 + j] = c2[:, j, :]
    for t in range(3):
        out[15 + t] = jnp.broadcast_to(s2[0, t], (BLK, CTX_DIM))
    for j in range(N_CLS_CTX):
        out[18 + j] = c3[:, j, :]
    for t in range(55):
        out[22 + t] = jnp.broadcast_to(suf[0, t], (BLK, CTX_DIM))


def _prompt_concat(label, cls_ctx, cls_ctx2, cls_ctx3, token_prefix,
                   token_suffix_1, token_suffix_2, token_suffix):
    c1, c2, c3 = _gather_sc(label.astype(jnp.int32), cls_ctx, cls_ctx2,
                            cls_ctx3)
    cblk = pl.BlockSpec((BLK, N_CLS_CTX, CTX_DIM), lambda i: (i, 0, 0))
    out_tm = pl.pallas_call(
        _assemble,
        grid=(B // BLK,),
        in_specs=[
            cblk,
            cblk,
            cblk,
            pl.BlockSpec((1, 5, CTX_DIM), lambda i: (0, 0, 0)),
            pl.BlockSpec((1, 2, CTX_DIM), lambda i: (0, 0, 0)),
            pl.BlockSpec((1, 3, CTX_DIM), lambda i: (0, 0, 0)),
            pl.BlockSpec((1, 55, CTX_DIM), lambda i: (0, 0, 0)),
        ],
        out_specs=pl.BlockSpec((N_TOK, BLK, CTX_DIM), lambda i: (0, i, 0)),
        out_shape=jax.ShapeDtypeStruct((N_TOK, B, CTX_DIM), jnp.float32),
        compiler_params=pltpu.CompilerParams(
            dimension_semantics=("arbitrary",)),
    )(c1, c2, c3, token_prefix, token_suffix_1, token_suffix_2, token_suffix)
    return out_tm.transpose(1, 0, 2)


def kernel(label, cls_ctx, cls_ctx2, cls_ctx3, token_prefix, token_suffix_1,
           token_suffix_2, token_suffix):
    return _prompt_concat(label, cls_ctx, cls_ctx2, cls_ctx3, token_prefix,
                          token_suffix_1, token_suffix_2, token_suffix)
